# baseline (speedup 1.0000x reference)
import functools

import jax
import jax.numpy as jnp
from jax import lax
from jax.experimental import pallas as pl
from jax.experimental.pallas import tpu as pltpu
from jax.experimental.pallas import tpu_sc as plsc

N = 10000
E = 320000
D = 128
R = 3
G = 64
EPS = 1e-5

HD = D // 2
NC = 2
NS = 16
CH = 80
EPT = E // NS
NCH = EPT // CH
NB = 5
BUFS = 2
ACC_ROWS = 30000
STRIPE = ACC_ROWS // NS

BN = 400
GRID = N // BN


def _sc_scatter_body(y0, y1, gidx_h, sidx_h, zacc_h, a_out,
                     gidx_v, sidx_v, rows, gsems, ssems, acc):
    c = lax.axis_index("c")
    s = lax.axis_index("s")
    pltpu.sync_copy(zacc_h.at[pl.ds(s * STRIPE, STRIPE)],
                    acc.at[pl.ds(s * STRIPE, STRIPE)])
    plsc.subcore_barrier()

    groups = [list(range(g, min(g + BUFS, NB))) for g in range(0, NB, BUFS)]

    def run(y):
        def outer(j2, carry):
            pltpu.sync_copy(gidx_h.at[s, pl.ds(j2 * NB, NB)], gidx_v)
            pltpu.sync_copy(sidx_h.at[s, pl.ds(j2 * NB, NB)], sidx_v)
            pend_g = {}
            for i in groups[0]:
                b = i % BUFS
                pend_g[i] = pltpu.async_copy(y.at[gidx_v.at[i]], rows.at[b],
                                             gsems.at[b])
            for gi, grp in enumerate(groups):
                scs = []
                for i in grp:
                    b = i % BUFS
                    pend_g.pop(i).wait()
                    scs.append((b, pltpu.async_copy(
                        rows.at[b], acc.at[sidx_v.at[i]], ssems.at[b],
                        add=True)))
                nxt = groups[gi + 1] if gi + 1 < len(groups) else []
                for j in nxt:
                    bj = j % BUFS
                    for b, sd in scs:
                        if b == bj:
                            sd.wait()
                            scs.remove((b, sd))
                            break
                    pend_g[j] = pltpu.async_copy(y.at[gidx_v.at[j]],
                                                 rows.at[bj], gsems.at[bj])
                for b, sd in scs:
                    sd.wait()
            return carry
        lax.fori_loop(0, NCH // NB, outer, 0)

    @pl.when(c == 0)
    def _():
        run(y0)

    @pl.when(c == 1)
    def _():
        run(y1)

    plsc.subcore_barrier()
    pltpu.sync_copy(acc.at[pl.ds(s * STRIPE, STRIPE)],
                    a_out.at[c, pl.ds(s * STRIPE, STRIPE)])


def _sc_mesh():
    return plsc.VectorSubcoreMesh(core_axis_name="c", subcore_axis_name="s")


def _scatter_call(y0f, y1f, gidx_t, sidx_t, zacc):
    fn = pl.kernel(
        _sc_scatter_body,
        out_type=jax.ShapeDtypeStruct((NC, ACC_ROWS, HD), jnp.float32),
        mesh=_sc_mesh(),
        scratch_types=[
            pltpu.VMEM((NB, CH), jnp.int32),
            pltpu.VMEM((NB, CH), jnp.int32),
            pltpu.VMEM((BUFS, CH, HD), jnp.float32),
            pltpu.SemaphoreType.DMA((BUFS,)),
            pltpu.SemaphoreType.DMA((BUFS,)),
            pltpu.VMEM_SHARED((ACC_ROWS, HD), jnp.float32),
        ],
        compiler_params=pltpu.CompilerParams(use_tc_tiling_on_sc=False),
    )
    return fn(y0f, y1f, gidx_t, sidx_t, zacc)


def _k1_body(h_ref, wrel_ref, wroot_ref, b_ref, y0_ref, y1_ref, z_ref):
    h = h_ref[...]
    z_ref[...] = jnp.dot(h, wroot_ref[...],
                         preferred_element_type=jnp.float32) + b_ref[...]
    for t in range(R):
        y = jnp.dot(h, wrel_ref[t], preferred_element_type=jnp.float32)
        y0_ref[t] = y[:, :HD]
        y1_ref[t] = y[:, HD:]


_k1 = pl.pallas_call(
    _k1_body,
    grid=(GRID,),
    in_specs=[
        pl.BlockSpec((BN, D), lambda i: (i, 0)),
        pl.BlockSpec((R, D, D), lambda i: (0, 0, 0)),
        pl.BlockSpec((D, D), lambda i: (0, 0)),
        pl.BlockSpec((1, D), lambda i: (0, 0)),
    ],
    out_specs=[
        pl.BlockSpec((R, BN, HD), lambda i: (0, i, 0)),
        pl.BlockSpec((R, BN, HD), lambda i: (0, i, 0)),
        pl.BlockSpec((BN, D), lambda i: (i, 0)),
    ],
    out_shape=[
        jax.ShapeDtypeStruct((R, N, HD), jnp.float32),
        jax.ShapeDtypeStruct((R, N, HD), jnp.float32),
        jax.ShapeDtypeStruct((N, D), jnp.float32),
    ],
)


def _k2_body(z_ref, a0_ref, a1_ref, a2_ref, c0_ref, c1_ref, c2_ref,
             hp_ref, st_ref):
    i = pl.program_id(0)
    h = z_ref[...]
    for a_ref, c_ref in ((a0_ref, c0_ref), (a1_ref, c1_ref), (a2_ref, c2_ref)):
        ab = a_ref[...]
        cb = c_ref[...]
        srow = jnp.concatenate([ab[0], ab[1]], axis=1)
        cnt = cb[0, :, 0:1]
        h = h + srow / jnp.maximum(cnt, 1.0)
    hp_ref[...] = h
    s1 = jnp.sum(h, axis=0, keepdims=True)
    s2 = jnp.sum(h * h, axis=0, keepdims=True)
    upd = jnp.concatenate([s1, s2, jnp.zeros((6, D), jnp.float32)], axis=0)

    @pl.when(i == 0)
    def _():
        st_ref[...] = upd

    @pl.when(i > 0)
    def _():
        st_ref[...] = st_ref[...] + upd


_k2 = pl.pallas_call(
    _k2_body,
    grid=(GRID,),
    in_specs=[pl.BlockSpec((BN, D), lambda i: (i, 0))]
    + [pl.BlockSpec((NC, BN, HD), lambda i, t=t: (0, t * GRID + i, 0))
       for t in range(R)]
    + [pl.BlockSpec((NC, BN, HD), lambda i, t=t: (0, t * GRID + i, 0))
       for t in range(R)],
    out_specs=[
        pl.BlockSpec((BN, D), lambda i: (i, 0)),
        pl.BlockSpec((8, D), lambda i: (0, 0)),
    ],
    out_shape=[
        jax.ShapeDtypeStruct((N, D), jnp.float32),
        jax.ShapeDtypeStruct((8, D), jnp.float32),
    ],
)


def _k3_body(hp_ref, st_ref, gb_ref, out_ref):
    st = st_ref[...]
    m = st[0:1] / float(N)
    ex2 = st[1:2] / float(N)
    inv = lax.rsqrt(ex2 - m * m + EPS)
    out_ref[...] = jnp.maximum(
        (hp_ref[...] - m) * inv * gb_ref[0:1] + gb_ref[1:2], 0.0)


_k3 = pl.pallas_call(
    _k3_body,
    grid=(GRID,),
    in_specs=[
        pl.BlockSpec((BN, D), lambda i: (i, 0)),
        pl.BlockSpec((8, D), lambda i: (0, 0)),
        pl.BlockSpec((2, D), lambda i: (0, 0)),
    ],
    out_specs=pl.BlockSpec((BN, D), lambda i: (i, 0)),
    out_shape=jax.ShapeDtypeStruct((N, D), jnp.float32),
)


def _k4_body(h_ref, bt_ref, out_ref, sums, cnts):
    i = pl.program_id(0)

    @pl.when(i == 0)
    def _():
        sums[...] = jnp.zeros((G, D), jnp.float32)
        cnts[...] = jnp.zeros((G, D), jnp.float32)

    bids = bt_ref[...][0, 0, :]
    oh = (bids[:, None] ==
          lax.broadcasted_iota(jnp.int32, (BN, G), 1)).astype(jnp.float32)
    dn = (((0,), (0,)), ((), ()))
    sums[...] += lax.dot_general(oh, h_ref[...], dn,
                                 preferred_element_type=jnp.float32)
    cnts[...] += lax.dot_general(oh, jnp.ones((BN, D), jnp.float32), dn,
                                 preferred_element_type=jnp.float32)

    @pl.when(i == GRID - 1)
    def _():
        out_ref[...] = sums[...] / jnp.maximum(cnts[...], 1.0)


_k4 = pl.pallas_call(
    _k4_body,
    grid=(GRID,),
    in_specs=[
        pl.BlockSpec((BN, D), lambda i: (i, 0)),
        pl.BlockSpec((1, 1, BN), lambda i: (i, 0, 0)),
    ],
    out_specs=pl.BlockSpec((G, D), lambda i: (0, 0)),
    out_shape=jax.ShapeDtypeStruct((G, D), jnp.float32),
    scratch_shapes=[
        pltpu.VMEM((G, D), jnp.float32),
        pltpu.VMEM((G, D), jnp.float32),
    ],
)


def kernel(x, edge_index, edge_type, batch,
           W_rel_0, W_root_0, b_0, gamma_0, beta_0,
           W_rel_1, W_root_1, b_1, gamma_1, beta_1,
           W_rel_2, W_root_2, b_2, gamma_2, beta_2):
    src = edge_index[0].astype(jnp.int32)
    dst = edge_index[1].astype(jnp.int32)
    et = edge_type.astype(jnp.int32)

    gidx = (et * N + src).reshape(NS, NCH, CH)
    sidx = (et * N + dst).reshape(NS, NCH, CH)

    zacc = jnp.zeros((ACC_ROWS, HD), jnp.float32)
    ones_y = jnp.ones((R * N, HD), jnp.float32)
    bt = batch.astype(jnp.int32).reshape(GRID, 1, BN)

    cnt_raw = _scatter_call(ones_y, ones_y, gidx, sidx, zacc)

    params = [
        (W_rel_0, W_root_0, b_0, gamma_0, beta_0),
        (W_rel_1, W_root_1, b_1, gamma_1, beta_1),
        (W_rel_2, W_root_2, b_2, gamma_2, beta_2),
    ]
    h = x
    for (W_rel, W_root, b, gamma, beta) in params:
        y0, y1, z = _k1(h, W_rel, W_root, b.reshape(1, D))
        a = _scatter_call(y0.reshape(R * N, HD), y1.reshape(R * N, HD),
                          gidx, sidx, zacc)
        hp, st = _k2(z, a, a, a, cnt_raw, cnt_raw, cnt_raw)
        h = _k3(hp, st, jnp.stack([gamma, beta]))
    return _k4(h, bt)


# speedup vs baseline: 1.0860x; 1.0860x over previous
import functools

import jax
import jax.numpy as jnp
from jax import lax
from jax.experimental import pallas as pl
from jax.experimental.pallas import tpu as pltpu
from jax.experimental.pallas import tpu_sc as plsc

N = 10000
E = 320000
D = 128
R = 3
G = 64
EPS = 1e-5

HD = D // 2
NC = 2
NS = 16
CH = 40
EPT = E // NS
NCH = EPT // CH
NB = 10
BUFS = 4
ACC_ROWS = 30000
STRIPE = ACC_ROWS // NS

BN = 400
GRID = N // BN


def _sc_scatter_body(y0, y1, gidx_h, sidx_h, zacc_h, a_out,
                     gidx_v, sidx_v, rows, gsems, ssems, acc):
    c = lax.axis_index("c")
    s = lax.axis_index("s")
    pltpu.sync_copy(zacc_h.at[pl.ds(s * STRIPE, STRIPE)],
                    acc.at[pl.ds(s * STRIPE, STRIPE)])
    plsc.subcore_barrier()

    groups = [list(range(g, min(g + BUFS, NB))) for g in range(0, NB, BUFS)]

    def run(y):
        def outer(j2, carry):
            pltpu.sync_copy(gidx_h.at[s, pl.ds(j2 * NB, NB)], gidx_v)
            pltpu.sync_copy(sidx_h.at[s, pl.ds(j2 * NB, NB)], sidx_v)
            pend_g = {}
            for i in groups[0]:
                b = i % BUFS
                pend_g[i] = pltpu.async_copy(y.at[gidx_v.at[i]], rows.at[b],
                                             gsems.at[b])
            for gi, grp in enumerate(groups):
                scs = []
                for i in grp:
                    b = i % BUFS
                    pend_g.pop(i).wait()
                    scs.append((b, pltpu.async_copy(
                        rows.at[b], acc.at[sidx_v.at[i]], ssems.at[b],
                        add=True)))
                nxt = groups[gi + 1] if gi + 1 < len(groups) else []
                for j in nxt:
                    bj = j % BUFS
                    for b, sd in scs:
                        if b == bj:
                            sd.wait()
                            scs.remove((b, sd))
                            break
                    pend_g[j] = pltpu.async_copy(y.at[gidx_v.at[j]],
                                                 rows.at[bj], gsems.at[bj])
                for b, sd in scs:
                    sd.wait()
            return carry
        lax.fori_loop(0, NCH // NB, outer, 0)

    @pl.when(c == 0)
    def _():
        run(y0)

    @pl.when(c == 1)
    def _():
        run(y1)

    plsc.subcore_barrier()
    pltpu.sync_copy(acc.at[pl.ds(s * STRIPE, STRIPE)],
                    a_out.at[c, pl.ds(s * STRIPE, STRIPE)])


def _sc_mesh():
    return plsc.VectorSubcoreMesh(core_axis_name="c", subcore_axis_name="s")


def _scatter_call(y0f, y1f, gidx_t, sidx_t, zacc):
    fn = pl.kernel(
        _sc_scatter_body,
        out_type=jax.ShapeDtypeStruct((NC, ACC_ROWS, HD), jnp.float32),
        mesh=_sc_mesh(),
        scratch_types=[
            pltpu.VMEM((NB, CH), jnp.int32),
            pltpu.VMEM((NB, CH), jnp.int32),
            pltpu.VMEM((BUFS, CH, HD), jnp.float32),
            pltpu.SemaphoreType.DMA((BUFS,)),
            pltpu.SemaphoreType.DMA((BUFS,)),
            pltpu.VMEM_SHARED((ACC_ROWS, HD), jnp.float32),
        ],
        compiler_params=pltpu.CompilerParams(use_tc_tiling_on_sc=False),
    )
    return fn(y0f, y1f, gidx_t, sidx_t, zacc)


def _k1_body(h_ref, wrel_ref, wroot_ref, b_ref, y0_ref, y1_ref, z_ref):
    h = h_ref[...]
    z_ref[...] = jnp.dot(h, wroot_ref[...],
                         preferred_element_type=jnp.float32) + b_ref[...]
    for t in range(R):
        y = jnp.dot(h, wrel_ref[t], preferred_element_type=jnp.float32)
        y0_ref[t] = y[:, :HD]
        y1_ref[t] = y[:, HD:]


_k1 = pl.pallas_call(
    _k1_body,
    grid=(GRID,),
    in_specs=[
        pl.BlockSpec((BN, D), lambda i: (i, 0)),
        pl.BlockSpec((R, D, D), lambda i: (0, 0, 0)),
        pl.BlockSpec((D, D), lambda i: (0, 0)),
        pl.BlockSpec((1, D), lambda i: (0, 0)),
    ],
    out_specs=[
        pl.BlockSpec((R, BN, HD), lambda i: (0, i, 0)),
        pl.BlockSpec((R, BN, HD), lambda i: (0, i, 0)),
        pl.BlockSpec((BN, D), lambda i: (i, 0)),
    ],
    out_shape=[
        jax.ShapeDtypeStruct((R, N, HD), jnp.float32),
        jax.ShapeDtypeStruct((R, N, HD), jnp.float32),
        jax.ShapeDtypeStruct((N, D), jnp.float32),
    ],
)


def _k2_body(z_ref, a0_ref, a1_ref, a2_ref, c0_ref, c1_ref, c2_ref,
             hp_ref, st_ref):
    i = pl.program_id(0)
    h = z_ref[...]
    for a_ref, c_ref in ((a0_ref, c0_ref), (a1_ref, c1_ref), (a2_ref, c2_ref)):
        ab = a_ref[...]
        cb = c_ref[...]
        srow = jnp.concatenate([ab[0], ab[1]], axis=1)
        cnt = cb[0, :, 0:1]
        h = h + srow / jnp.maximum(cnt, 1.0)
    hp_ref[...] = h
    s1 = jnp.sum(h, axis=0, keepdims=True)
    s2 = jnp.sum(h * h, axis=0, keepdims=True)
    upd = jnp.concatenate([s1, s2, jnp.zeros((6, D), jnp.float32)], axis=0)

    @pl.when(i == 0)
    def _():
        st_ref[...] = upd

    @pl.when(i > 0)
    def _():
        st_ref[...] = st_ref[...] + upd


_k2 = pl.pallas_call(
    _k2_body,
    grid=(GRID,),
    in_specs=[pl.BlockSpec((BN, D), lambda i: (i, 0))]
    + [pl.BlockSpec((NC, BN, HD), lambda i, t=t: (0, t * GRID + i, 0))
       for t in range(R)]
    + [pl.BlockSpec((NC, BN, HD), lambda i, t=t: (0, t * GRID + i, 0))
       for t in range(R)],
    out_specs=[
        pl.BlockSpec((BN, D), lambda i: (i, 0)),
        pl.BlockSpec((8, D), lambda i: (0, 0)),
    ],
    out_shape=[
        jax.ShapeDtypeStruct((N, D), jnp.float32),
        jax.ShapeDtypeStruct((8, D), jnp.float32),
    ],
)


def _k3_body(hp_ref, st_ref, gb_ref, out_ref):
    st = st_ref[...]
    m = st[0:1] / float(N)
    ex2 = st[1:2] / float(N)
    inv = lax.rsqrt(ex2 - m * m + EPS)
    out_ref[...] = jnp.maximum(
        (hp_ref[...] - m) * inv * gb_ref[0:1] + gb_ref[1:2], 0.0)


_k3 = pl.pallas_call(
    _k3_body,
    grid=(GRID,),
    in_specs=[
        pl.BlockSpec((BN, D), lambda i: (i, 0)),
        pl.BlockSpec((8, D), lambda i: (0, 0)),
        pl.BlockSpec((2, D), lambda i: (0, 0)),
    ],
    out_specs=pl.BlockSpec((BN, D), lambda i: (i, 0)),
    out_shape=jax.ShapeDtypeStruct((N, D), jnp.float32),
)


def _k4_body(h_ref, bt_ref, out_ref, sums, cnts):
    i = pl.program_id(0)

    @pl.when(i == 0)
    def _():
        sums[...] = jnp.zeros((G, D), jnp.float32)
        cnts[...] = jnp.zeros((G, D), jnp.float32)

    bids = bt_ref[...][0, 0, :]
    oh = (bids[:, None] ==
          lax.broadcasted_iota(jnp.int32, (BN, G), 1)).astype(jnp.float32)
    dn = (((0,), (0,)), ((), ()))
    sums[...] += lax.dot_general(oh, h_ref[...], dn,
                                 preferred_element_type=jnp.float32)
    cnts[...] += lax.dot_general(oh, jnp.ones((BN, D), jnp.float32), dn,
                                 preferred_element_type=jnp.float32)

    @pl.when(i == GRID - 1)
    def _():
        out_ref[...] = sums[...] / jnp.maximum(cnts[...], 1.0)


_k4 = pl.pallas_call(
    _k4_body,
    grid=(GRID,),
    in_specs=[
        pl.BlockSpec((BN, D), lambda i: (i, 0)),
        pl.BlockSpec((1, 1, BN), lambda i: (i, 0, 0)),
    ],
    out_specs=pl.BlockSpec((G, D), lambda i: (0, 0)),
    out_shape=jax.ShapeDtypeStruct((G, D), jnp.float32),
    scratch_shapes=[
        pltpu.VMEM((G, D), jnp.float32),
        pltpu.VMEM((G, D), jnp.float32),
    ],
)


def kernel(x, edge_index, edge_type, batch,
           W_rel_0, W_root_0, b_0, gamma_0, beta_0,
           W_rel_1, W_root_1, b_1, gamma_1, beta_1,
           W_rel_2, W_root_2, b_2, gamma_2, beta_2):
    src = edge_index[0].astype(jnp.int32)
    dst = edge_index[1].astype(jnp.int32)
    et = edge_type.astype(jnp.int32)

    gidx = (et * N + src).reshape(NS, NCH, CH)
    sidx = (et * N + dst).reshape(NS, NCH, CH)

    zacc = jnp.zeros((ACC_ROWS, HD), jnp.float32)
    ones_y = jnp.ones((R * N, HD), jnp.float32)
    bt = batch.astype(jnp.int32).reshape(GRID, 1, BN)

    cnt_raw = _scatter_call(ones_y, ones_y, gidx, sidx, zacc)

    params = [
        (W_rel_0, W_root_0, b_0, gamma_0, beta_0),
        (W_rel_1, W_root_1, b_1, gamma_1, beta_1),
        (W_rel_2, W_root_2, b_2, gamma_2, beta_2),
    ]
    h = x
    for (W_rel, W_root, b, gamma, beta) in params:
        y0, y1, z = _k1(h, W_rel, W_root, b.reshape(1, D))
        a = _scatter_call(y0.reshape(R * N, HD), y1.reshape(R * N, HD),
                          gidx, sidx, zacc)
        hp, st = _k2(z, a, a, a, cnt_raw, cnt_raw, cnt_raw)
        h = _k3(hp, st, jnp.stack([gamma, beta]))
    return _k4(h, bt)


# speedup vs baseline: 1.1430x; 1.0525x over previous
import functools

import jax
import jax.numpy as jnp
from jax import lax
from jax.experimental import pallas as pl
from jax.experimental.pallas import tpu as pltpu
from jax.experimental.pallas import tpu_sc as plsc

N = 10000
E = 320000
D = 128
R = 3
G = 64
EPS = 1e-5

HD = D // 2
NC = 2
NS = 16
CH = 40
EPT = E // NS
NCH = EPT // CH
NB = 20
BUFS = 3
ACC_ROWS = 30000
STRIPE = ACC_ROWS // NS

BN = 400
GRID = N // BN


def _sc_scatter_body(y0, y1, gidx_h, sidx_h, zacc_h, a_out,
                     gidx_v, sidx_v, rows, gsems, ssems, acc):
    c = lax.axis_index("c")
    s = lax.axis_index("s")
    pltpu.sync_copy(zacc_h.at[pl.ds(s * STRIPE, STRIPE)],
                    acc.at[pl.ds(s * STRIPE, STRIPE)])
    plsc.subcore_barrier()

    groups = [list(range(g, min(g + BUFS, NB))) for g in range(0, NB, BUFS)]

    def run(y):
        def outer(j2, carry):
            pltpu.sync_copy(gidx_h.at[s, pl.ds(j2 * NB, NB)], gidx_v)
            pltpu.sync_copy(sidx_h.at[s, pl.ds(j2 * NB, NB)], sidx_v)
            pend_g = {}
            for i in groups[0]:
                b = i % BUFS
                pend_g[i] = pltpu.async_copy(y.at[gidx_v.at[i]], rows.at[b],
                                             gsems.at[b])
            for gi, grp in enumerate(groups):
                scs = []
                for i in grp:
                    b = i % BUFS
                    pend_g.pop(i).wait()
                    scs.append((b, pltpu.async_copy(
                        rows.at[b], acc.at[sidx_v.at[i]], ssems.at[b],
                        add=True)))
                nxt = groups[gi + 1] if gi + 1 < len(groups) else []
                for j in nxt:
                    bj = j % BUFS
                    for b, sd in scs:
                        if b == bj:
                            sd.wait()
                            scs.remove((b, sd))
                            break
                    pend_g[j] = pltpu.async_copy(y.at[gidx_v.at[j]],
                                                 rows.at[bj], gsems.at[bj])
                for b, sd in scs:
                    sd.wait()
            return carry
        lax.fori_loop(0, NCH // NB, outer, 0)

    @pl.when(c == 0)
    def _():
        run(y0)

    @pl.when(c == 1)
    def _():
        run(y1)

    plsc.subcore_barrier()
    pltpu.sync_copy(acc.at[pl.ds(s * STRIPE, STRIPE)],
                    a_out.at[c, pl.ds(s * STRIPE, STRIPE)])


def _sc_mesh():
    return plsc.VectorSubcoreMesh(core_axis_name="c", subcore_axis_name="s")


def _scatter_call(y0f, y1f, gidx_t, sidx_t, zacc):
    fn = pl.kernel(
        _sc_scatter_body,
        out_type=jax.ShapeDtypeStruct((NC, ACC_ROWS, HD), jnp.float32),
        mesh=_sc_mesh(),
        scratch_types=[
            pltpu.VMEM((NB, CH), jnp.int32),
            pltpu.VMEM((NB, CH), jnp.int32),
            pltpu.VMEM((BUFS, CH, HD), jnp.float32),
            pltpu.SemaphoreType.DMA((BUFS,)),
            pltpu.SemaphoreType.DMA((BUFS,)),
            pltpu.VMEM_SHARED((ACC_ROWS, HD), jnp.float32),
        ],
        compiler_params=pltpu.CompilerParams(use_tc_tiling_on_sc=False),
    )
    return fn(y0f, y1f, gidx_t, sidx_t, zacc)


def _k1_body(h_ref, wrel_ref, wroot_ref, b_ref, y0_ref, y1_ref, z_ref):
    h = h_ref[...]
    z_ref[...] = jnp.dot(h, wroot_ref[...],
                         preferred_element_type=jnp.float32) + b_ref[...]
    for t in range(R):
        y = jnp.dot(h, wrel_ref[t], preferred_element_type=jnp.float32)
        y0_ref[t] = y[:, :HD]
        y1_ref[t] = y[:, HD:]


_k1 = pl.pallas_call(
    _k1_body,
    grid=(GRID,),
    in_specs=[
        pl.BlockSpec((BN, D), lambda i: (i, 0)),
        pl.BlockSpec((R, D, D), lambda i: (0, 0, 0)),
        pl.BlockSpec((D, D), lambda i: (0, 0)),
        pl.BlockSpec((1, D), lambda i: (0, 0)),
    ],
    out_specs=[
        pl.BlockSpec((R, BN, HD), lambda i: (0, i, 0)),
        pl.BlockSpec((R, BN, HD), lambda i: (0, i, 0)),
        pl.BlockSpec((BN, D), lambda i: (i, 0)),
    ],
    out_shape=[
        jax.ShapeDtypeStruct((R, N, HD), jnp.float32),
        jax.ShapeDtypeStruct((R, N, HD), jnp.float32),
        jax.ShapeDtypeStruct((N, D), jnp.float32),
    ],
)


def _k2_body(z_ref, a0_ref, a1_ref, a2_ref, c0_ref, c1_ref, c2_ref,
             hp_ref, st_ref):
    i = pl.program_id(0)
    h = z_ref[...]
    for a_ref, c_ref in ((a0_ref, c0_ref), (a1_ref, c1_ref), (a2_ref, c2_ref)):
        ab = a_ref[...]
        cb = c_ref[...]
        srow = jnp.concatenate([ab[0], ab[1]], axis=1)
        cnt = cb[0, :, 0:1]
        h = h + srow / jnp.maximum(cnt, 1.0)
    hp_ref[...] = h
    s1 = jnp.sum(h, axis=0, keepdims=True)
    s2 = jnp.sum(h * h, axis=0, keepdims=True)
    upd = jnp.concatenate([s1, s2, jnp.zeros((6, D), jnp.float32)], axis=0)

    @pl.when(i == 0)
    def _():
        st_ref[...] = upd

    @pl.when(i > 0)
    def _():
        st_ref[...] = st_ref[...] + upd


_k2 = pl.pallas_call(
    _k2_body,
    grid=(GRID,),
    in_specs=[pl.BlockSpec((BN, D), lambda i: (i, 0))]
    + [pl.BlockSpec((NC, BN, HD), lambda i, t=t: (0, t * GRID + i, 0))
       for t in range(R)]
    + [pl.BlockSpec((NC, BN, HD), lambda i, t=t: (0, t * GRID + i, 0))
       for t in range(R)],
    out_specs=[
        pl.BlockSpec((BN, D), lambda i: (i, 0)),
        pl.BlockSpec((8, D), lambda i: (0, 0)),
    ],
    out_shape=[
        jax.ShapeDtypeStruct((N, D), jnp.float32),
        jax.ShapeDtypeStruct((8, D), jnp.float32),
    ],
)


def _bn_relu(hp_ref, st_ref, gb_ref):
    st = st_ref[...]
    m = st[0:1] / float(N)
    ex2 = st[1:2] / float(N)
    inv = lax.rsqrt(ex2 - m * m + EPS)
    return jnp.maximum((hp_ref[...] - m) * inv * gb_ref[0:1] + gb_ref[1:2],
                       0.0)


def _k1f_body(hp_ref, st_ref, gb_ref, wrel_ref, wroot_ref, b_ref,
              y0_ref, y1_ref, z_ref):
    h = _bn_relu(hp_ref, st_ref, gb_ref)
    z_ref[...] = jnp.dot(h, wroot_ref[...],
                         preferred_element_type=jnp.float32) + b_ref[...]
    for t in range(R):
        y = jnp.dot(h, wrel_ref[t], preferred_element_type=jnp.float32)
        y0_ref[t] = y[:, :HD]
        y1_ref[t] = y[:, HD:]


_k1f = pl.pallas_call(
    _k1f_body,
    grid=(GRID,),
    in_specs=[
        pl.BlockSpec((BN, D), lambda i: (i, 0)),
        pl.BlockSpec((8, D), lambda i: (0, 0)),
        pl.BlockSpec((2, D), lambda i: (0, 0)),
        pl.BlockSpec((R, D, D), lambda i: (0, 0, 0)),
        pl.BlockSpec((D, D), lambda i: (0, 0)),
        pl.BlockSpec((1, D), lambda i: (0, 0)),
    ],
    out_specs=[
        pl.BlockSpec((R, BN, HD), lambda i: (0, i, 0)),
        pl.BlockSpec((R, BN, HD), lambda i: (0, i, 0)),
        pl.BlockSpec((BN, D), lambda i: (i, 0)),
    ],
    out_shape=[
        jax.ShapeDtypeStruct((R, N, HD), jnp.float32),
        jax.ShapeDtypeStruct((R, N, HD), jnp.float32),
        jax.ShapeDtypeStruct((N, D), jnp.float32),
    ],
)


def _k4_body(hp_ref, st_ref, gb_ref, bt_ref, out_ref, sums, cnts):
    i = pl.program_id(0)

    @pl.when(i == 0)
    def _():
        sums[...] = jnp.zeros((G, D), jnp.float32)
        cnts[...] = jnp.zeros((G, D), jnp.float32)

    h = _bn_relu(hp_ref, st_ref, gb_ref)
    bids = bt_ref[...][0, 0, :]
    oh = (bids[:, None] ==
          lax.broadcasted_iota(jnp.int32, (BN, G), 1)).astype(jnp.float32)
    dn = (((0,), (0,)), ((), ()))
    sums[...] += lax.dot_general(oh, h, dn,
                                 preferred_element_type=jnp.float32)
    cnts[...] += lax.dot_general(oh, jnp.ones((BN, D), jnp.float32), dn,
                                 preferred_element_type=jnp.float32)

    @pl.when(i == GRID - 1)
    def _():
        out_ref[...] = sums[...] / jnp.maximum(cnts[...], 1.0)


_k4 = pl.pallas_call(
    _k4_body,
    grid=(GRID,),
    in_specs=[
        pl.BlockSpec((BN, D), lambda i: (i, 0)),
        pl.BlockSpec((8, D), lambda i: (0, 0)),
        pl.BlockSpec((2, D), lambda i: (0, 0)),
        pl.BlockSpec((1, 1, BN), lambda i: (i, 0, 0)),
    ],
    out_specs=pl.BlockSpec((G, D), lambda i: (0, 0)),
    out_shape=jax.ShapeDtypeStruct((G, D), jnp.float32),
    scratch_shapes=[
        pltpu.VMEM((G, D), jnp.float32),
        pltpu.VMEM((G, D), jnp.float32),
    ],
)


def kernel(x, edge_index, edge_type, batch,
           W_rel_0, W_root_0, b_0, gamma_0, beta_0,
           W_rel_1, W_root_1, b_1, gamma_1, beta_1,
           W_rel_2, W_root_2, b_2, gamma_2, beta_2):
    src = edge_index[0].astype(jnp.int32)
    dst = edge_index[1].astype(jnp.int32)
    et = edge_type.astype(jnp.int32)

    gidx = (et * N + src).reshape(NS, NCH, CH)
    sidx = (et * N + dst).reshape(NS, NCH, CH)

    zacc = jnp.zeros((ACC_ROWS, HD), jnp.float32)
    ones_y = jnp.ones((R * N, HD), jnp.float32)
    bt = batch.astype(jnp.int32).reshape(GRID, 1, BN)

    cnt_raw = _scatter_call(ones_y, ones_y, gidx, sidx, zacc)

    params = [
        (W_rel_0, W_root_0, b_0, gamma_0, beta_0),
        (W_rel_1, W_root_1, b_1, gamma_1, beta_1),
        (W_rel_2, W_root_2, b_2, gamma_2, beta_2),
    ]
    hp, st, gb = None, None, None
    for li, (W_rel, W_root, b, gamma, beta) in enumerate(params):
        if li == 0:
            y0, y1, z = _k1(x, W_rel, W_root, b.reshape(1, D))
        else:
            y0, y1, z = _k1f(hp, st, gb, W_rel, W_root, b.reshape(1, D))
        a = _scatter_call(y0.reshape(R * N, HD), y1.reshape(R * N, HD),
                          gidx, sidx, zacc)
        hp, st = _k2(z, a, a, a, cnt_raw, cnt_raw, cnt_raw)
        gb = jnp.stack([gamma, beta])
    return _k4(hp, st, gb, bt)
